# baseline (speedup 1.0000x reference)
import functools

import jax
import jax.numpy as jnp
from jax import lax
from jax.experimental import pallas as pl
from jax.experimental.pallas import tpu as pltpu
from jax.experimental.pallas import tpu_sc as plsc

_DIMS = (119, 4, 12, 12, 10, 6, 6, 2, 2)
_H = 128
_N = 100000
_NT = 3
_TBL = 2432
_NW = 32
_C = 80
_CHUNKS = 39
_ROWS_MAIN = _C * _CHUNKS
_EXTRA0 = _ROWS_MAIN * _NW
_N_EXTRA = (_N - _EXTRA0) // 16
_PAIRS = (_CHUNKS - 1) // 2


def _sc_encode(tbl, xw):
    mesh = plsc.VectorSubcoreMesh(core_axis_name="c", subcore_axis_name="s")

    @functools.partial(
        pl.kernel,
        mesh=mesh,
        out_type=jax.ShapeDtypeStruct((_N, _H), jnp.float32),
        scratch_types=[
            pltpu.VMEM_SHARED((_TBL, _H), jnp.float32),
            pltpu.VMEM((_ROWS_MAIN + 16,), jnp.int32),
            pltpu.VMEM((_C, _H), jnp.float32),
            pltpu.VMEM((_C, _H), jnp.float32),
            pltpu.VMEM((_C, _H), jnp.float32),
            pltpu.VMEM((_C, _H), jnp.float32),
            pltpu.VMEM((512, _H), jnp.float32),
            pltpu.VMEM((_C,), jnp.int32),
            pltpu.VMEM((_C,), jnp.int32),
            pltpu.SemaphoreType.DMA,
            pltpu.SemaphoreType.DMA,
            pltpu.SemaphoreType.DMA,
            pltpu.SemaphoreType.DMA,
        ],
        compiler_params=pltpu.CompilerParams(needs_layout_passes=False),
    )
    def body(tbl_hbm, xw_hbm, out_hbm, tbl_sh, x_v, st0, st1, ob0, ob1,
             t23_v, iA0, iA1, ssem0, ssem1, osem0, osem1):
        sid = lax.axis_index("s")
        wid = lax.axis_index("c") * 16 + sid

        rows_per_tile = _TBL // 16
        pltpu.sync_copy(
            tbl_hbm.at[pl.ds(sid * rows_per_tile, rows_per_tile), :],
            tbl_sh.at[pl.ds(sid * rows_per_tile, rows_per_tile), :],
        )
        plsc.subcore_barrier()
        pltpu.sync_copy(tbl_hbm.at[pl.ds(1904, 512), :], t23_v)

        row0 = wid * _ROWS_MAIN
        pltpu.sync_copy(
            xw_hbm.at[pl.ds(row0, _ROWS_MAIN)], x_v.at[pl.ds(0, _ROWS_MAIN)]
        )

        def issue(xbase, st, iA, ssem):
            for j in range(_C // 16):
                w = x_v[pl.ds(xbase + j * 16, 16)]
                iA[pl.ds(j * 16, 16)] = w & 0x7FF
            pltpu.async_copy(tbl_sh.at[iA], st, ssem)

        def wait_streams(st, ssem):
            pltpu.make_async_copy(
                tbl_hbm.at[pl.ds(0, _C), :], st, ssem
            ).wait()

        def sum_chunk(xbase, st, ob):
            def srow(r, carry):
                w = x_v[pl.ds(xbase + r, 16)][0]
                rB = (w >> 11) & 0xFF
                rC = (w >> 19) + 144
                for cb in range(8):
                    a = st[r, pl.ds(cb * 16, 16)]
                    a = a + t23_v[rB, pl.ds(cb * 16, 16)]
                    a = a + t23_v[rC, pl.ds(cb * 16, 16)]
                    ob[r, pl.ds(cb * 16, 16)] = a
                return carry

            lax.fori_loop(0, _C, srow, 0, unroll=4)

        def issue_out(ob, out_row, osem):
            pltpu.async_copy(ob, out_hbm.at[pl.ds(out_row, _C), :], osem)

        def wait_out(ob, osem):
            pltpu.make_async_copy(
                ob, out_hbm.at[pl.ds(0, _C), :], osem
            ).wait()

        issue(0, st0, iA0, ssem0)

        def pair(i, carry):
            c0 = 2 * i
            issue((c0 + 1) * _C, st1, iA1, ssem1)
            wait_streams(st0, ssem0)

            @pl.when(i > 0)
            def _w0():
                wait_out(ob0, osem0)

            sum_chunk(c0 * _C, st0, ob0)
            issue_out(ob0, row0 + c0 * _C, osem0)
            issue((c0 + 2) * _C, st0, iA0, ssem0)
            wait_streams(st1, ssem1)

            @pl.when(i > 0)
            def _w1():
                wait_out(ob1, osem1)

            sum_chunk((c0 + 1) * _C, st1, ob1)
            issue_out(ob1, row0 + (c0 + 1) * _C, osem1)
            return carry

        lax.fori_loop(0, _PAIRS, pair, 0)

        wait_streams(st0, ssem0)
        wait_out(ob0, osem0)
        sum_chunk((_CHUNKS - 1) * _C, st0, ob0)
        issue_out(ob0, row0 + (_CHUNKS - 1) * _C, osem0)

        @pl.when(wid < _N_EXTRA)
        def _extra():
            erow = _EXTRA0 + wid * 16
            pltpu.sync_copy(xw_hbm.at[pl.ds(erow, 16)], x_v.at[pl.ds(0, 16)])
            w = x_v[pl.ds(0, 16)]
            pltpu.async_copy(
                tbl_sh.at[w & 0x7FF], st1.at[pl.ds(0, 16), :], ssem1
            )
            pltpu.make_async_copy(
                tbl_hbm.at[pl.ds(0, 16), :],
                st1.at[pl.ds(0, 16), :],
                ssem1,
            ).wait()
            wait_out(ob1, osem1)

            def erowsum(r, carry):
                ww = x_v[pl.ds(r, 16)][0]
                rB = (ww >> 11) & 0xFF
                rC = (ww >> 19) + 144
                for cb in range(8):
                    a = st1[r, pl.ds(cb * 16, 16)]
                    a = a + t23_v[rB, pl.ds(cb * 16, 16)]
                    a = a + t23_v[rC, pl.ds(cb * 16, 16)]
                    ob1[r, pl.ds(cb * 16, 16)] = a
                return carry

            lax.fori_loop(0, 16, erowsum, 0, unroll=4)
            pltpu.async_copy(ob1.at[pl.ds(0, 16), :],
                             out_hbm.at[pl.ds(erow, 16), :], osem1)

        wait_out(ob0, osem0)

        @pl.when(wid < _N_EXTRA)
        def _drain_extra():
            pltpu.make_async_copy(
                ob1.at[pl.ds(0, 16), :], out_hbm.at[pl.ds(0, 16), :], osem1
            ).wait()

        @pl.when(wid >= _N_EXTRA)
        def _drain_noextra():
            wait_out(ob1, osem1)

    return body(tbl, xw)


def _fuse_tables(W0, W1, W2, W3, W4, W5, W6, W7, W8):
    t1 = (
        W8[:, None, None, None, :]
        + W7[None, :, None, None, :]
        + W1[None, None, :, None, :]
        + W0[None, None, None, :, :]
    ).reshape(1904, _H)
    t2 = (W3[:, None, :] + W2[None, :, :]).reshape(144, _H)
    t3 = (
        W6[:, None, None, :] + W5[None, :, None, :] + W4[None, None, :, :]
    ).reshape(360, _H)
    pad = jnp.zeros((24, _H), jnp.float32)
    return jnp.concatenate([t1, t2, t3, pad], axis=0)


def _pack_indices(x):
    x = x.astype(jnp.int32)
    c = [jnp.clip(x[:, i], 0, _DIMS[i] - 1) for i in range(9)]
    iA = c[0] + c[1] * 119 + c[7] * 476 + c[8] * 952
    iB = c[2] + c[3] * 12
    iC = c[4] + c[5] * 10 + c[6] * 60
    return iA | (iB << 11) | (iC << 19)


def kernel(x, W0, W1, W2, W3, W4, W5, W6, W7, W8):
    tbl = _fuse_tables(W0, W1, W2, W3, W4, W5, W6, W7, W8)
    return _sc_encode(tbl, _pack_indices(x))


# speedup vs baseline: 1.0634x; 1.0634x over previous
import functools

import jax
import jax.numpy as jnp
from jax import lax
from jax.experimental import pallas as pl
from jax.experimental.pallas import tpu as pltpu
from jax.experimental.pallas import tpu_sc as plsc

_DIMS = (119, 4, 12, 12, 10, 6, 6, 2, 2)
_H = 128
_N = 100000
_NT = 3
_TBL = 2432
_NW = 32
_C = 80
_CHUNKS = 39
_ROWS_MAIN = _C * _CHUNKS
_EXTRA0 = _ROWS_MAIN * _NW
_N_EXTRA = (_N - _EXTRA0) // 16
_PAIRS = (_CHUNKS - 1) // 2


def _sc_encode(tbl, xw):
    mesh = plsc.VectorSubcoreMesh(core_axis_name="c", subcore_axis_name="s")

    @functools.partial(
        pl.kernel,
        mesh=mesh,
        out_type=jax.ShapeDtypeStruct((_N, _H), jnp.float32),
        scratch_types=[
            pltpu.VMEM_SHARED((_TBL, _H), jnp.float32),
            pltpu.VMEM((_ROWS_MAIN + 16,), jnp.int32),
            pltpu.VMEM((_C, _H), jnp.float32),
            pltpu.VMEM((_C, _H), jnp.float32),
            pltpu.VMEM((_C, _H), jnp.float32),
            pltpu.VMEM((_C, _H), jnp.float32),
            pltpu.VMEM((512, _H), jnp.float32),
            pltpu.VMEM((_C,), jnp.int32),
            pltpu.VMEM((_C,), jnp.int32),
            pltpu.SemaphoreType.DMA,
            pltpu.SemaphoreType.DMA,
            pltpu.SemaphoreType.DMA,
            pltpu.SemaphoreType.DMA,
        ],
        compiler_params=pltpu.CompilerParams(needs_layout_passes=False),
    )
    def body(tbl_hbm, xw_hbm, out_hbm, tbl_sh, x_v, st0, st1, ob0, ob1,
             t23_v, iA0, iA1, ssem0, ssem1, osem0, osem1):
        sid = lax.axis_index("s")
        wid = lax.axis_index("c") * 16 + sid

        rows_per_tile = _TBL // 16
        pltpu.sync_copy(
            tbl_hbm.at[pl.ds(sid * rows_per_tile, rows_per_tile), :],
            tbl_sh.at[pl.ds(sid * rows_per_tile, rows_per_tile), :],
        )
        plsc.subcore_barrier()
        pltpu.sync_copy(tbl_hbm.at[pl.ds(1904, 512), :], t23_v)

        row0 = wid * _ROWS_MAIN
        pltpu.sync_copy(
            xw_hbm.at[pl.ds(row0, _ROWS_MAIN)], x_v.at[pl.ds(0, _ROWS_MAIN)]
        )

        def issue(xbase, st, iA, ssem):
            for j in range(_C // 16):
                w = x_v[pl.ds(xbase + j * 16, 16)]
                iA[pl.ds(j * 16, 16)] = w & 0x7FF
            pltpu.async_copy(tbl_sh.at[iA], st, ssem)

        def wait_streams(st, ssem):
            pltpu.make_async_copy(
                tbl_hbm.at[pl.ds(0, _C), :], st, ssem
            ).wait()

        lanes = lax.iota(jnp.int32, 16)
        civ = [lanes + cb * 16 for cb in range(8)]

        def sum_chunk(xbase, st, ob):
            def srow(r, carry):
                w16 = plsc.load_gather(
                    x_v, [jnp.zeros((16,), jnp.int32) + (xbase + r)]
                )
                rB = (w16 >> 11) & 0xFF
                rC = (w16 >> 19) + 144
                for cb in range(8):
                    a = st[r, pl.ds(cb * 16, 16)]
                    a = a + plsc.load_gather(t23_v, [rB, civ[cb]])
                    a = a + plsc.load_gather(t23_v, [rC, civ[cb]])
                    ob[r, pl.ds(cb * 16, 16)] = a
                return carry

            lax.fori_loop(0, _C, srow, 0, unroll=4)

        def issue_out(ob, out_row, osem):
            pltpu.async_copy(ob, out_hbm.at[pl.ds(out_row, _C), :], osem)

        def wait_out(ob, osem):
            pltpu.make_async_copy(
                ob, out_hbm.at[pl.ds(0, _C), :], osem
            ).wait()

        issue(0, st0, iA0, ssem0)

        def pair(i, carry):
            c0 = 2 * i
            issue((c0 + 1) * _C, st1, iA1, ssem1)
            wait_streams(st0, ssem0)

            @pl.when(i > 0)
            def _w0():
                wait_out(ob0, osem0)

            sum_chunk(c0 * _C, st0, ob0)
            issue_out(ob0, row0 + c0 * _C, osem0)
            issue((c0 + 2) * _C, st0, iA0, ssem0)
            wait_streams(st1, ssem1)

            @pl.when(i > 0)
            def _w1():
                wait_out(ob1, osem1)

            sum_chunk((c0 + 1) * _C, st1, ob1)
            issue_out(ob1, row0 + (c0 + 1) * _C, osem1)
            return carry

        lax.fori_loop(0, _PAIRS, pair, 0)

        wait_streams(st0, ssem0)
        wait_out(ob0, osem0)
        sum_chunk((_CHUNKS - 1) * _C, st0, ob0)
        issue_out(ob0, row0 + (_CHUNKS - 1) * _C, osem0)

        @pl.when(wid < _N_EXTRA)
        def _extra():
            erow = _EXTRA0 + wid * 16
            pltpu.sync_copy(xw_hbm.at[pl.ds(erow, 16)], x_v.at[pl.ds(0, 16)])
            w = x_v[pl.ds(0, 16)]
            pltpu.async_copy(
                tbl_sh.at[w & 0x7FF], st1.at[pl.ds(0, 16), :], ssem1
            )
            pltpu.make_async_copy(
                tbl_hbm.at[pl.ds(0, 16), :],
                st1.at[pl.ds(0, 16), :],
                ssem1,
            ).wait()
            wait_out(ob1, osem1)

            def erowsum(r, carry):
                w16 = plsc.load_gather(
                    x_v, [jnp.zeros((16,), jnp.int32) + r]
                )
                rB = (w16 >> 11) & 0xFF
                rC = (w16 >> 19) + 144
                for cb in range(8):
                    a = st1[r, pl.ds(cb * 16, 16)]
                    a = a + plsc.load_gather(t23_v, [rB, civ[cb]])
                    a = a + plsc.load_gather(t23_v, [rC, civ[cb]])
                    ob1[r, pl.ds(cb * 16, 16)] = a
                return carry

            lax.fori_loop(0, 16, erowsum, 0, unroll=4)
            pltpu.async_copy(ob1.at[pl.ds(0, 16), :],
                             out_hbm.at[pl.ds(erow, 16), :], osem1)

        wait_out(ob0, osem0)

        @pl.when(wid < _N_EXTRA)
        def _drain_extra():
            pltpu.make_async_copy(
                ob1.at[pl.ds(0, 16), :], out_hbm.at[pl.ds(0, 16), :], osem1
            ).wait()

        @pl.when(wid >= _N_EXTRA)
        def _drain_noextra():
            wait_out(ob1, osem1)

    return body(tbl, xw)


def _fuse_tables(W0, W1, W2, W3, W4, W5, W6, W7, W8):
    t1 = (
        W8[:, None, None, None, :]
        + W7[None, :, None, None, :]
        + W1[None, None, :, None, :]
        + W0[None, None, None, :, :]
    ).reshape(1904, _H)
    t2 = (W3[:, None, :] + W2[None, :, :]).reshape(144, _H)
    t3 = (
        W6[:, None, None, :] + W5[None, :, None, :] + W4[None, None, :, :]
    ).reshape(360, _H)
    pad = jnp.zeros((24, _H), jnp.float32)
    return jnp.concatenate([t1, t2, t3, pad], axis=0)


def _pack_indices(x):
    x = x.astype(jnp.int32)
    c = [jnp.clip(x[:, i], 0, _DIMS[i] - 1) for i in range(9)]
    iA = c[0] + c[1] * 119 + c[7] * 476 + c[8] * 952
    iB = c[2] + c[3] * 12
    iC = c[4] + c[5] * 10 + c[6] * 60
    return iA | (iB << 11) | (iC << 19)


def kernel(x, W0, W1, W2, W3, W4, W5, W6, W7, W8):
    tbl = _fuse_tables(W0, W1, W2, W3, W4, W5, W6, W7, W8)
    return _sc_encode(tbl, _pack_indices(x))


# speedup vs baseline: 1.5767x; 1.4827x over previous
import functools

import jax
import jax.numpy as jnp
from jax import lax
from jax.experimental import pallas as pl
from jax.experimental.pallas import tpu as pltpu
from jax.experimental.pallas import tpu_sc as plsc

_DIMS = (119, 4, 12, 12, 10, 6, 6, 2, 2)
_H = 128
_N = 100000
_NT = 3
_TBL = 2432
_NW = 32
_C = 80
_CHUNKS = 39
_ROWS_MAIN = _C * _CHUNKS
_EXTRA0 = _ROWS_MAIN * _NW
_N_EXTRA = (_N - _EXTRA0) // 16
_PAIRS = (_CHUNKS - 1) // 2


def _sc_encode(tbl, xw):
    mesh = plsc.VectorSubcoreMesh(core_axis_name="c", subcore_axis_name="s")

    @functools.partial(
        pl.kernel,
        mesh=mesh,
        out_type=jax.ShapeDtypeStruct((_N, _H), jnp.float32),
        scratch_types=[
            pltpu.VMEM_SHARED((_TBL, _H), jnp.float32),
            pltpu.VMEM((_ROWS_MAIN,), jnp.int32),
            pltpu.VMEM((_NT * _C, _H), jnp.float32),
            pltpu.VMEM((_NT * _C, _H), jnp.float32),
            pltpu.VMEM((_C, _H), jnp.float32),
            pltpu.VMEM((_C, _H), jnp.float32),
            pltpu.VMEM((_C,), jnp.int32),
            pltpu.VMEM((_C,), jnp.int32),
            pltpu.VMEM((_C,), jnp.int32),
            pltpu.VMEM((_C,), jnp.int32),
            pltpu.VMEM((_C,), jnp.int32),
            pltpu.VMEM((_C,), jnp.int32),
            pltpu.SemaphoreType.DMA,
            pltpu.SemaphoreType.DMA,
            pltpu.SemaphoreType.DMA,
            pltpu.SemaphoreType.DMA,
        ],
        compiler_params=pltpu.CompilerParams(needs_layout_passes=False),
    )
    def body(tbl_hbm, xw_hbm, out_hbm, tbl_sh, x_v, st0, st1, ob0, ob1,
             iA0, iB0, iC0, iA1, iB1, iC1, ssem0, ssem1, osem0, osem1):
        sid = lax.axis_index("s")
        wid = lax.axis_index("c") * 16 + sid

        rows_per_tile = _TBL // 16
        pltpu.sync_copy(
            tbl_hbm.at[pl.ds(sid * rows_per_tile, rows_per_tile), :],
            tbl_sh.at[pl.ds(sid * rows_per_tile, rows_per_tile), :],
        )
        plsc.subcore_barrier()

        row0 = wid * _ROWS_MAIN
        pltpu.sync_copy(xw_hbm.at[pl.ds(row0, _ROWS_MAIN)], x_v)

        def issue(xbase, st, iA, iB, iC, ssem):
            for j in range(_C // 16):
                w = x_v[pl.ds(xbase + j * 16, 16)]
                iA[pl.ds(j * 16, 16)] = w & 0x7FF
                iB[pl.ds(j * 16, 16)] = ((w >> 11) & 0xFF) + 1904
                iC[pl.ds(j * 16, 16)] = (w >> 19) + 2048
            for t, idx in enumerate((iA, iB, iC)):
                pltpu.async_copy(
                    tbl_sh.at[idx], st.at[pl.ds(t * _C, _C), :], ssem
                )

        def wait_streams(st, ssem):
            pltpu.make_async_copy(
                tbl_hbm.at[pl.ds(0, _NT * _C), :], st, ssem
            ).wait()

        def sum_chunk(st, ob):
            def srow(r, carry):
                for cb in range(8):
                    a = st[r, pl.ds(cb * 16, 16)]
                    for t in range(1, _NT):
                        a = a + st[t * _C + r, pl.ds(cb * 16, 16)]
                    ob[r, pl.ds(cb * 16, 16)] = a
                return carry

            lax.fori_loop(0, _C, srow, 0, unroll=4)

        def issue_out(ob, out_row, osem):
            pltpu.async_copy(ob, out_hbm.at[pl.ds(out_row, _C), :], osem)

        def wait_out(ob, osem):
            pltpu.make_async_copy(
                ob, out_hbm.at[pl.ds(0, _C), :], osem
            ).wait()

        issue(0, st0, iA0, iB0, iC0, ssem0)

        def pair(i, carry):
            c0 = 2 * i
            issue((c0 + 1) * _C, st1, iA1, iB1, iC1, ssem1)
            wait_streams(st0, ssem0)

            @pl.when(i > 0)
            def _w0():
                wait_out(ob0, osem0)

            sum_chunk(st0, ob0)
            issue_out(ob0, row0 + c0 * _C, osem0)
            issue((c0 + 2) * _C, st0, iA0, iB0, iC0, ssem0)
            wait_streams(st1, ssem1)

            @pl.when(i > 0)
            def _w1():
                wait_out(ob1, osem1)

            sum_chunk(st1, ob1)
            issue_out(ob1, row0 + (c0 + 1) * _C, osem1)
            return carry

        lax.fori_loop(0, _PAIRS, pair, 0)

        wait_streams(st0, ssem0)
        wait_out(ob0, osem0)
        sum_chunk(st0, ob0)
        issue_out(ob0, row0 + (_CHUNKS - 1) * _C, osem0)

        @pl.when(wid < _N_EXTRA)
        def _extra():
            erow = _EXTRA0 + wid * 16
            pltpu.sync_copy(xw_hbm.at[pl.ds(erow, 16)], x_v.at[pl.ds(0, 16)])
            w = x_v[pl.ds(0, 16)]
            eidx = (w & 0x7FF, ((w >> 11) & 0xFF) + 1904, (w >> 19) + 2048)
            for t in range(_NT):
                pltpu.async_copy(
                    tbl_sh.at[eidx[t]], st1.at[pl.ds(t * _C, 16), :], ssem1
                )
            pltpu.make_async_copy(
                tbl_hbm.at[pl.ds(0, _NT * 16), :],
                st1.at[pl.ds(0, _NT * 16), :],
                ssem1,
            ).wait()
            wait_out(ob1, osem1)

            def erowsum(r, carry):
                for cb in range(8):
                    a = st1[r, pl.ds(cb * 16, 16)]
                    for t in range(1, _NT):
                        a = a + st1[t * _C + r, pl.ds(cb * 16, 16)]
                    ob1[r, pl.ds(cb * 16, 16)] = a
                return carry

            lax.fori_loop(0, 16, erowsum, 0, unroll=4)
            pltpu.async_copy(ob1.at[pl.ds(0, 16), :],
                             out_hbm.at[pl.ds(erow, 16), :], osem1)

        wait_out(ob0, osem0)

        @pl.when(wid < _N_EXTRA)
        def _drain_extra():
            pltpu.make_async_copy(
                ob1.at[pl.ds(0, 16), :], out_hbm.at[pl.ds(0, 16), :], osem1
            ).wait()

        @pl.when(wid >= _N_EXTRA)
        def _drain_noextra():
            wait_out(ob1, osem1)

    return body(tbl, xw)


def _fuse_tables(W0, W1, W2, W3, W4, W5, W6, W7, W8):
    t1 = (
        W8[:, None, None, None, :]
        + W7[None, :, None, None, :]
        + W1[None, None, :, None, :]
        + W0[None, None, None, :, :]
    ).reshape(1904, _H)
    t2 = (W3[:, None, :] + W2[None, :, :]).reshape(144, _H)
    t3 = (
        W6[:, None, None, :] + W5[None, :, None, :] + W4[None, None, :, :]
    ).reshape(360, _H)
    pad = jnp.zeros((24, _H), jnp.float32)
    return jnp.concatenate([t1, t2, t3, pad], axis=0)


def _pack_indices(x):
    x = x.astype(jnp.int32)
    c = [jnp.clip(x[:, i], 0, _DIMS[i] - 1) for i in range(9)]
    iA = c[0] + c[1] * 119 + c[7] * 476 + c[8] * 952
    iB = c[2] + c[3] * 12
    iC = c[4] + c[5] * 10 + c[6] * 60
    return iA | (iB << 11) | (iC << 19)


def kernel(x, W0, W1, W2, W3, W4, W5, W6, W7, W8):
    tbl = _fuse_tables(W0, W1, W2, W3, W4, W5, W6, W7, W8)
    return _sc_encode(tbl, _pack_indices(x))


# speedup vs baseline: 1.6854x; 1.0690x over previous
import functools

import jax
import jax.numpy as jnp
from jax import lax
from jax.experimental import pallas as pl
from jax.experimental.pallas import tpu as pltpu
from jax.experimental.pallas import tpu_sc as plsc

_DIMS = (119, 4, 12, 12, 10, 6, 6, 2, 2)
_H = 128
_N = 100000
_NT = 3
_TBL = 2432
_NW = 32
_C = 104
_CHUNKS = 30
_ROWS_MAIN = _C * _CHUNKS
_EXTRA0 = _ROWS_MAIN * _NW
_N_EXTRA = (_N - _EXTRA0) // 16
_PAIRS = (_CHUNKS - 1) // 2


def _sc_encode(tbl, xw):
    mesh = plsc.VectorSubcoreMesh(core_axis_name="c", subcore_axis_name="s")

    @functools.partial(
        pl.kernel,
        mesh=mesh,
        out_type=jax.ShapeDtypeStruct((_N, _H), jnp.float32),
        scratch_types=[
            pltpu.VMEM_SHARED((_TBL, _H), jnp.float32),
            pltpu.VMEM((_ROWS_MAIN,), jnp.int32),
            pltpu.VMEM((_NT * _C, _H), jnp.float32),
            pltpu.VMEM((_NT * _C, _H), jnp.float32),
            pltpu.VMEM((_C, _H), jnp.float32),
            pltpu.VMEM((_C, _H), jnp.float32),
            pltpu.VMEM((_C,), jnp.int32),
            pltpu.VMEM((_C,), jnp.int32),
            pltpu.VMEM((_C,), jnp.int32),
            pltpu.VMEM((_C,), jnp.int32),
            pltpu.VMEM((_C,), jnp.int32),
            pltpu.VMEM((_C,), jnp.int32),
            pltpu.SemaphoreType.DMA,
            pltpu.SemaphoreType.DMA,
            pltpu.SemaphoreType.DMA,
            pltpu.SemaphoreType.DMA,
        ],
        compiler_params=pltpu.CompilerParams(needs_layout_passes=False),
    )
    def body(tbl_hbm, xw_hbm, out_hbm, tbl_sh, x_v, st0, st1, ob0, ob1,
             iA0, iB0, iC0, iA1, iB1, iC1, ssem0, ssem1, osem0, osem1):
        sid = lax.axis_index("s")
        wid = lax.axis_index("c") * 16 + sid

        rows_per_tile = _TBL // 16
        pltpu.sync_copy(
            tbl_hbm.at[pl.ds(sid * rows_per_tile, rows_per_tile), :],
            tbl_sh.at[pl.ds(sid * rows_per_tile, rows_per_tile), :],
        )
        plsc.subcore_barrier()

        row0 = wid * _ROWS_MAIN
        pltpu.sync_copy(xw_hbm.at[pl.ds(row0, _ROWS_MAIN)], x_v)

        def issue(xbase, st, iA, iB, iC, ssem):
            for j in range(_C // 16):
                w = x_v[pl.ds(xbase + j * 16, 16)]
                iA[pl.ds(j * 16, 16)] = w & 0x7FF
                iB[pl.ds(j * 16, 16)] = ((w >> 11) & 0xFF) + 1904
                iC[pl.ds(j * 16, 16)] = (w >> 19) + 2048
            for t, idx in enumerate((iA, iB, iC)):
                pltpu.async_copy(
                    tbl_sh.at[idx], st.at[pl.ds(t * _C, _C), :], ssem
                )

        def wait_streams(st, ssem):
            pltpu.make_async_copy(
                tbl_hbm.at[pl.ds(0, _NT * _C), :], st, ssem
            ).wait()

        def sum_chunk(st, ob):
            def srow(r, carry):
                for cb in range(8):
                    a = st[r, pl.ds(cb * 16, 16)]
                    for t in range(1, _NT):
                        a = a + st[t * _C + r, pl.ds(cb * 16, 16)]
                    ob[r, pl.ds(cb * 16, 16)] = a
                return carry

            lax.fori_loop(0, _C, srow, 0, unroll=4)

        def issue_out(ob, out_row, osem):
            pltpu.async_copy(ob, out_hbm.at[pl.ds(out_row, _C), :], osem)

        def wait_out(ob, osem):
            pltpu.make_async_copy(
                ob, out_hbm.at[pl.ds(0, _C), :], osem
            ).wait()

        issue(0, st0, iA0, iB0, iC0, ssem0)

        def pair(i, carry):
            c0 = 2 * i
            issue((c0 + 1) * _C, st1, iA1, iB1, iC1, ssem1)
            wait_streams(st0, ssem0)

            @pl.when(i > 0)
            def _w0():
                wait_out(ob0, osem0)

            sum_chunk(st0, ob0)
            issue_out(ob0, row0 + c0 * _C, osem0)
            issue((c0 + 2) * _C, st0, iA0, iB0, iC0, ssem0)
            wait_streams(st1, ssem1)

            @pl.when(i > 0)
            def _w1():
                wait_out(ob1, osem1)

            sum_chunk(st1, ob1)
            issue_out(ob1, row0 + (c0 + 1) * _C, osem1)
            return carry

        lax.fori_loop(0, _PAIRS, pair, 0)

        wait_streams(st0, ssem0)
        wait_out(ob0, osem0)
        sum_chunk(st0, ob0)
        issue_out(ob0, row0 + (_CHUNKS - 1) * _C, osem0)

        @pl.when(wid < _N_EXTRA)
        def _extra():
            erow = _EXTRA0 + wid * 16
            pltpu.sync_copy(xw_hbm.at[pl.ds(erow, 16)], x_v.at[pl.ds(0, 16)])
            w = x_v[pl.ds(0, 16)]
            eidx = (w & 0x7FF, ((w >> 11) & 0xFF) + 1904, (w >> 19) + 2048)
            for t in range(_NT):
                pltpu.async_copy(
                    tbl_sh.at[eidx[t]], st1.at[pl.ds(t * _C, 16), :], ssem1
                )
            pltpu.make_async_copy(
                tbl_hbm.at[pl.ds(0, _NT * 16), :],
                st1.at[pl.ds(0, _NT * 16), :],
                ssem1,
            ).wait()
            wait_out(ob1, osem1)

            def erowsum(r, carry):
                for cb in range(8):
                    a = st1[r, pl.ds(cb * 16, 16)]
                    for t in range(1, _NT):
                        a = a + st1[t * _C + r, pl.ds(cb * 16, 16)]
                    ob1[r, pl.ds(cb * 16, 16)] = a
                return carry

            lax.fori_loop(0, 16, erowsum, 0, unroll=4)
            pltpu.async_copy(ob1.at[pl.ds(0, 16), :],
                             out_hbm.at[pl.ds(erow, 16), :], osem1)

        wait_out(ob0, osem0)

        @pl.when(wid < _N_EXTRA)
        def _drain_extra():
            pltpu.make_async_copy(
                ob1.at[pl.ds(0, 16), :], out_hbm.at[pl.ds(0, 16), :], osem1
            ).wait()

        @pl.when(wid >= _N_EXTRA)
        def _drain_noextra():
            wait_out(ob1, osem1)

    return body(tbl, xw)


def _fuse_tables(W0, W1, W2, W3, W4, W5, W6, W7, W8):
    t1 = (
        W8[:, None, None, None, :]
        + W7[None, :, None, None, :]
        + W1[None, None, :, None, :]
        + W0[None, None, None, :, :]
    ).reshape(1904, _H)
    t2 = (W3[:, None, :] + W2[None, :, :]).reshape(144, _H)
    t3 = (
        W6[:, None, None, :] + W5[None, :, None, :] + W4[None, None, :, :]
    ).reshape(360, _H)
    pad = jnp.zeros((24, _H), jnp.float32)
    return jnp.concatenate([t1, t2, t3, pad], axis=0)


def _pack_indices(x):
    x = x.astype(jnp.int32)
    c = [jnp.clip(x[:, i], 0, _DIMS[i] - 1) for i in range(9)]
    iA = c[0] + c[1] * 119 + c[7] * 476 + c[8] * 952
    iB = c[2] + c[3] * 12
    iC = c[4] + c[5] * 10 + c[6] * 60
    return iA | (iB << 11) | (iC << 19)


def kernel(x, W0, W1, W2, W3, W4, W5, W6, W7, W8):
    tbl = _fuse_tables(W0, W1, W2, W3, W4, W5, W6, W7, W8)
    return _sc_encode(tbl, _pack_indices(x))
